# baseline (speedup 1.0000x reference)
import functools

import jax
import jax.numpy as jnp
from jax.experimental import pallas as pl
from jax.experimental.pallas import tpu as pltpu

TOPK = 8
NEG = -jnp.inf


def _body(o_ref, t_ref, tw_ref, out_ref, *, bb, k, hw, nblocks):
    i = pl.program_id(0)
    diff = o_ref[...] - t_ref[...]
    sums = jnp.sum(diff * diff, axis=2)
    tw = tw_ref[...]
    losses = sums * (tw * tw) * (1.0 / hw)

    kiota = jax.lax.broadcasted_iota(jnp.int32, (bb, k), 1)
    acc = jnp.zeros((bb,), jnp.float32)
    vals = losses
    for _ in range(TOPK):
        m = jnp.max(vals, axis=1)
        acc = acc + m
        eq = vals == m[:, None]
        first = jnp.min(jnp.where(eq, kiota, k), axis=1)
        vals = jnp.where(kiota == first[:, None], NEG, vals)

    part = jnp.sum(acc)

    @pl.when(i == 0)
    def _():
        out_ref[0, 0] = 0.0

    out_ref[0, 0] += part


def kernel(output, target, target_weights):
    b, k, h, w = output.shape
    hw = h * w
    o3 = output.reshape(b, k, hw)
    t3 = target.reshape(b, k, hw)
    bb = 16
    nblocks = b // bb
    f = pl.pallas_call(
        functools.partial(_body, bb=bb, k=k, hw=hw, nblocks=nblocks),
        grid=(nblocks,),
        in_specs=[
            pl.BlockSpec((bb, k, hw), lambda i: (i, 0, 0)),
            pl.BlockSpec((bb, k, hw), lambda i: (i, 0, 0)),
            pl.BlockSpec((bb, k), lambda i: (i, 0)),
        ],
        out_specs=pl.BlockSpec((1, 1), lambda i: (0, 0), memory_space=pltpu.SMEM),
        out_shape=jax.ShapeDtypeStruct((1, 1), jnp.float32),
    )
    total = f(o3, t3, target_weights)
    return (total[0, 0] / (b * TOPK)).astype(jnp.float32)
